# Initial kernel scaffold; baseline (speedup 1.0000x reference)
import jax
import jax.numpy as jnp
from jax.experimental import pallas as pl


def kernel(outputs, targets):
    raise NotImplementedError("write your pallas kernel here")


# speedup vs baseline: 5.6920x; 5.6920x over previous
import functools

import jax
import jax.numpy as jnp
from jax import lax
from jax.experimental import pallas as pl
from jax.experimental.pallas import tpu as pltpu
from jax.experimental.pallas import tpu_sc as plsc

_B = 128
_V = 100000
_NC = 2
_NW = 32
_L = 16
_RPW = 4
_C = 400
_NCH = _V // _C
_INF = float("inf")


def _make_sc_call():
    mesh = plsc.VectorSubcoreMesh(core_axis_name="c", subcore_axis_name="s")

    @functools.partial(
        pl.kernel,
        mesh=mesh,
        compiler_params=pltpu.CompilerParams(needs_layout_passes=False),
        out_type=jax.ShapeDtypeStruct((_NW, _L), jnp.float32),
        scratch_types=[
            pltpu.VMEM((_B,), jnp.int32),
            pltpu.VMEM((_L, _B), jnp.float32),
            pltpu.VMEM((_C, _B), jnp.float32),
            pltpu.VMEM((_L,), jnp.float32),
            pltpu.SemaphoreType.DMA,
        ],
    )
    def sc_accuracy(out_t_hbm, targets_hbm, out_hbm, tgt_v, gbuf_v, buf_v,
                    outv, sem):
        gid = lax.axis_index("s") * _NC + lax.axis_index("c")
        blk = gid // _RPW
        quar = gid - blk * _RPW
        col0 = pl.multiple_of(blk * _L, _L)
        lanes = lax.iota(jnp.int32, _L)
        lo = quar * _RPW
        relm = (lanes >= lo) & (lanes < lo + _RPW)
        ones = jnp.ones((_L,), jnp.float32)
        zeros = jnp.zeros((_L,), jnp.float32)

        pltpu.sync_copy(targets_hbm, tgt_v)

        def extract(vec, lane):
            sel = jnp.where(lanes == lane, vec, jnp.zeros_like(vec))
            return jnp.sum(sel)

        idxvec = tgt_v[pl.ds(col0, _L)]
        pltpu.async_copy(out_t_hbm.at[idxvec], gbuf_v, sem).wait()

        svec = jnp.where(relm, zeros, jnp.full((_L,), _INF))
        for r in range(_RPW):
            l = lo + r
            s_r = extract(gbuf_v[l, pl.ds(col0, _L)], l)
            svec = svec + jnp.where(lanes == l, s_r, zeros)
        tcols = tgt_v[pl.ds(col0, _L)]

        def chunk_cond(c):
            ci, cnt = c
            alive = jnp.sum(jnp.where(relm & (cnt < 5.0), ones, zeros))
            return (ci < _NCH) & (alive > 0.0)

        def chunk_body(c):
            ci, cnt = c
            voff = pl.multiple_of(ci * _C, 8)
            pltpu.sync_copy(out_t_hbm.at[pl.ds(voff, _C)], buf_v)

            def inner(i, acc):
                b = i * 4
                for j in range(4):
                    v = buf_v[b + j, pl.ds(col0, _L)]
                    acc = acc + jnp.where(v > svec, ones, zeros)
                return acc

            cnt = lax.fori_loop(0, _C // 4, inner, cnt)
            return ci + 1, cnt

        _, cnts = lax.while_loop(chunk_cond, chunk_body,
                                 (jnp.int32(0), zeros))

        need_eq = jnp.sum(jnp.where(relm & (cnts < 5.0), ones, zeros))

        def eq_pass(_):
            def eq_body(c):
                ci, eq = c
                voff = pl.multiple_of(ci * _C, 8)
                pltpu.sync_copy(out_t_hbm.at[pl.ds(voff, _C)], buf_v)

                def inner(i, acc):
                    b = i * 4
                    for j in range(4):
                        vg = voff + b + j
                        v = buf_v[b + j, pl.ds(col0, _L)]
                        m = (v == svec) & (vg < tcols)
                        acc = acc + jnp.where(m, ones, zeros)
                    return acc

                eq = lax.fori_loop(0, _C // 4, inner, eq)
                return ci + 1, eq

            _, eq = lax.while_loop(lambda c: c[0] < _NCH, eq_body,
                                   (jnp.int32(0), zeros))
            return eq

        eqs = lax.cond(need_eq > 0.0, eq_pass, lambda _: zeros, None)

        rank = cnts + eqs
        top1 = jnp.sum(jnp.where(relm & (rank < 1.0), ones, zeros))
        top5 = jnp.sum(jnp.where(relm & (rank < 5.0), ones, zeros))

        outv[...] = jnp.where(lanes == 0, top1,
                              jnp.where(lanes == 1, top5, 0.0))
        pltpu.sync_copy(outv, out_hbm.at[gid])

    return sc_accuracy


_sc_accuracy = _make_sc_call()


def kernel(outputs, targets):
    parts = _sc_accuracy(outputs.T, targets.astype(jnp.int32))
    s = parts.sum(axis=0) * (100.0 / _B)
    return (s[0:1], s[1:2])


# speedup vs baseline: 5.7389x; 1.0082x over previous
import functools

import jax
import jax.numpy as jnp
from jax import lax
from jax.experimental import pallas as pl
from jax.experimental.pallas import tpu as pltpu
from jax.experimental.pallas import tpu_sc as plsc

_B = 128
_V = 100000
_NC = 2
_NW = 32
_L = 16
_RPW = 4
_C = 400
_NCH = _V // _C
_INF = float("inf")


def _make_sc_call():
    mesh = plsc.VectorSubcoreMesh(core_axis_name="c", subcore_axis_name="s")

    @functools.partial(
        pl.kernel,
        mesh=mesh,
        compiler_params=pltpu.CompilerParams(needs_layout_passes=False),
        out_type=jax.ShapeDtypeStruct((_NW, _L), jnp.float32),
        scratch_types=[
            pltpu.VMEM((_B,), jnp.int32),
            pltpu.VMEM((_L, _B), jnp.float32),
            pltpu.VMEM((_C, _B), jnp.float32),
            pltpu.VMEM((_L,), jnp.float32),
            pltpu.SemaphoreType.DMA,
            pltpu.SemaphoreType.DMA,
        ],
    )
    def sc_accuracy(out_t_hbm, targets_hbm, out_hbm, tgt_v, gbuf_v, buf_v,
                    outv, sem, sem0):
        gid = lax.axis_index("s") * _NC + lax.axis_index("c")
        blk = gid // _RPW
        quar = gid - blk * _RPW
        col0 = pl.multiple_of(blk * _L, _L)
        lanes = lax.iota(jnp.int32, _L)
        lo = quar * _RPW
        relm = (lanes >= lo) & (lanes < lo + _RPW)
        ones = jnp.ones((_L,), jnp.float32)
        zeros = jnp.zeros((_L,), jnp.float32)

        cp_t = pltpu.async_copy(targets_hbm, tgt_v, sem)
        cp0 = pltpu.async_copy(out_t_hbm.at[pl.ds(0, _C)], buf_v, sem0)
        cp_t.wait()

        def extract(vec, lane):
            sel = jnp.where(lanes == lane, vec, jnp.zeros_like(vec))
            return jnp.sum(sel)

        idxvec = tgt_v[pl.ds(col0, _L)]
        pltpu.async_copy(out_t_hbm.at[idxvec], gbuf_v, sem).wait()

        svec = jnp.where(relm, zeros, jnp.full((_L,), _INF))
        for r in range(_RPW):
            l = lo + r
            s_r = extract(gbuf_v[l, pl.ds(col0, _L)], l)
            svec = svec + jnp.where(lanes == l, s_r, zeros)
        tcols = tgt_v[pl.ds(col0, _L)]
        cp0.wait()

        def chunk_cond(c):
            ci, cnt, _ = c
            alive = jnp.sum(jnp.where(relm & (cnt < 5.0), ones, zeros))
            return (ci < _NCH) & (alive > 0.0)

        def chunk_body(c):
            ci, cnt, eq = c
            voff = pl.multiple_of(ci * _C, 8)

            @pl.when(ci > 0)
            def _():
                pltpu.sync_copy(out_t_hbm.at[pl.ds(voff, _C)], buf_v)

            def inner(i, acc):
                cnt, eq = acc
                b = i * 4
                for j in range(4):
                    v = buf_v[b + j, pl.ds(col0, _L)]
                    cnt = cnt + jnp.where(v > svec, ones, zeros)
                    m = (v == svec) & (voff + b + j < tcols)
                    eq = eq + jnp.where(m, ones, zeros)
                return cnt, eq

            cnt, eq = lax.fori_loop(0, _C // 4, inner, (cnt, eq))
            return ci + 1, cnt, eq

        _, cnts, eqs = lax.while_loop(chunk_cond, chunk_body,
                                      (jnp.int32(0), zeros, zeros))

        rank = cnts + eqs
        top1 = jnp.sum(jnp.where(relm & (rank < 1.0), ones, zeros))
        top5 = jnp.sum(jnp.where(relm & (rank < 5.0), ones, zeros))

        outv[...] = jnp.where(lanes == 0, top1,
                              jnp.where(lanes == 1, top5, 0.0))
        pltpu.sync_copy(outv, out_hbm.at[gid])

    return sc_accuracy


_sc_accuracy = _make_sc_call()


def kernel(outputs, targets):
    parts = _sc_accuracy(outputs.T, targets.astype(jnp.int32))
    s = parts.sum(axis=0) * (100.0 / _B)
    return (s[0:1], s[1:2])


# speedup vs baseline: 6.0449x; 1.0533x over previous
import functools

import jax
import jax.numpy as jnp
from jax import lax
from jax.experimental import pallas as pl
from jax.experimental.pallas import tpu as pltpu
from jax.experimental.pallas import tpu_sc as plsc

_B = 128
_V = 100000
_NC = 2
_NW = 32
_L = 16
_RPW = 4
_C = 200
_NCH = _V // _C
_INF = float("inf")


def _make_sc_call():
    mesh = plsc.VectorSubcoreMesh(core_axis_name="c", subcore_axis_name="s")

    @functools.partial(
        pl.kernel,
        mesh=mesh,
        compiler_params=pltpu.CompilerParams(needs_layout_passes=False),
        out_type=jax.ShapeDtypeStruct((_NW, _L), jnp.float32),
        scratch_types=[
            pltpu.VMEM((_B,), jnp.int32),
            pltpu.VMEM((_L, _B), jnp.float32),
            pltpu.VMEM((_C, _B), jnp.float32),
            pltpu.VMEM((_L,), jnp.float32),
            pltpu.SemaphoreType.DMA,
            pltpu.SemaphoreType.DMA,
        ],
    )
    def sc_accuracy(out_t_hbm, targets_hbm, out_hbm, tgt_v, gbuf_v, buf_v,
                    outv, sem, sem0):
        gid = lax.axis_index("s") * _NC + lax.axis_index("c")
        blk = gid // _RPW
        quar = gid - blk * _RPW
        col0 = pl.multiple_of(blk * _L, _L)
        lanes = lax.iota(jnp.int32, _L)
        lo = quar * _RPW
        relm = (lanes >= lo) & (lanes < lo + _RPW)
        ones = jnp.ones((_L,), jnp.float32)
        zeros = jnp.zeros((_L,), jnp.float32)

        cp_t = pltpu.async_copy(targets_hbm, tgt_v, sem)
        cp0 = pltpu.async_copy(out_t_hbm.at[pl.ds(0, _C)], buf_v, sem0)
        cp_t.wait()

        def extract(vec, lane):
            sel = jnp.where(lanes == lane, vec, jnp.zeros_like(vec))
            return jnp.sum(sel)

        idxvec = tgt_v[pl.ds(col0, _L)]
        pltpu.async_copy(out_t_hbm.at[idxvec], gbuf_v, sem).wait()

        svec = jnp.where(relm, zeros, jnp.full((_L,), _INF))
        for r in range(_RPW):
            l = lo + r
            s_r = extract(gbuf_v[l, pl.ds(col0, _L)], l)
            svec = svec + jnp.where(lanes == l, s_r, zeros)
        tcols = tgt_v[pl.ds(col0, _L)]
        cp0.wait()

        def chunk_cond(c):
            ci, cnt, _ = c
            alive = jnp.sum(jnp.where(relm & (cnt < 5.0), ones, zeros))
            return (ci < _NCH) & (alive > 0.0)

        def chunk_body(c):
            ci, cnt, eq = c
            voff = pl.multiple_of(ci * _C, 8)

            @pl.when(ci > 0)
            def _():
                pltpu.sync_copy(out_t_hbm.at[pl.ds(voff, _C)], buf_v)

            def inner(i, acc):
                cnt, eq = acc
                b = i * 4
                for j in range(4):
                    v = buf_v[b + j, pl.ds(col0, _L)]
                    cnt = cnt + jnp.where(v > svec, ones, zeros)
                    m = (v == svec) & (voff + b + j < tcols)
                    eq = eq + jnp.where(m, ones, zeros)
                return cnt, eq

            cnt, eq = lax.fori_loop(0, _C // 4, inner, (cnt, eq))
            return ci + 1, cnt, eq

        _, cnts, eqs = lax.while_loop(chunk_cond, chunk_body,
                                      (jnp.int32(0), zeros, zeros))

        rank = cnts + eqs
        top1 = jnp.sum(jnp.where(relm & (rank < 1.0), ones, zeros))
        top5 = jnp.sum(jnp.where(relm & (rank < 5.0), ones, zeros))

        outv[...] = jnp.where(lanes == 0, top1,
                              jnp.where(lanes == 1, top5, 0.0))
        pltpu.sync_copy(outv, out_hbm.at[gid])

    return sc_accuracy


_sc_accuracy = _make_sc_call()


def kernel(outputs, targets):
    parts = _sc_accuracy(outputs.T, targets.astype(jnp.int32))
    s = parts.sum(axis=0) * (100.0 / _B)
    return (s[0:1], s[1:2])
